# baseline (speedup 1.0000x reference)
import functools

import jax
import jax.numpy as jnp
from jax import lax
from jax.experimental import pallas as pl
from jax.experimental.pallas import tpu as pltpu
from jax.experimental.pallas import tpu_sc as plsc

NC = 2
NS = 16
K = 40
G = 5
KA = 112
NBUF = 3
CW = 128


PH = 4


def _agg_chunks(e):
    nw = NC * NS
    cpw = -(-e // (nw * KA))
    return (cpw + 8 * PH - 1) // (8 * PH) * (8 * PH)


def _make_agg(n, e, d):
    cpw = _agg_chunks(e)
    cpp = cpw // PH
    nf = 10
    rpt = n // nf
    mesh = plsc.VectorSubcoreMesh(core_axis_name="c", subcore_axis_name="s")

    @functools.partial(
        pl.kernel,
        out_type=jax.ShapeDtypeStruct((NC * n, d), jnp.float32),
        mesh=mesh,
        scratch_types=[
            pltpu.VMEM_SHARED((n + 8, d), jnp.float32),
            pltpu.VMEM((cpp, KA), jnp.int32),
            pltpu.VMEM((cpp, KA), jnp.int32),
            pltpu.VMEM((NBUF, KA, d), jnp.float32),
            pltpu.SemaphoreType.DMA,
        ],
    )
    def agg(msg_hbm, srcp_hbm, dstp_hbm, zeros_hbm, out_hbm,
            acc_sh, sidx, didx, rows, semg):
        cid = lax.axis_index("c")
        sid = lax.axis_index("s")
        wid = sid * NC + cid
        @pl.when(sid < nf)
        def _():
            pltpu.sync_copy(zeros_hbm, acc_sh.at[pl.ds(sid * rpt, rpt)])
        plsc.subcore_barrier()

        for p in range(PH):
            row0 = wid * cpw + p * cpp
            pltpu.sync_copy(srcp_hbm.at[pl.ds(row0, cpp)], sidx)
            pltpu.sync_copy(dstp_hbm.at[pl.ds(row0, cpp)], didx)

            for b in range(NBUF):
                pltpu.async_copy(msg_hbm.at[sidx.at[b]], rows.at[b], semg)

            def outer(i, _):
                for b in range(NBUF):
                    c = i * NBUF + b
                    pltpu.make_async_copy(
                        msg_hbm.at[sidx.at[c]], rows.at[b], semg).wait()
                    pltpu.sync_copy(rows.at[b], acc_sh.at[didx.at[c]],
                                    add=True)
                    nxt = c + NBUF

                    @pl.when(nxt < cpp)
                    def _():
                        pltpu.async_copy(msg_hbm.at[sidx.at[nxt]],
                                         rows.at[b], semg)
                return 0

            lax.fori_loop(0, cpp // NBUF, outer, 0)

        plsc.subcore_barrier()

        @pl.when(sid < nf)
        def _():
            pltpu.sync_copy(acc_sh.at[pl.ds(sid * rpt, rpt)],
                            out_hbm.at[pl.ds(cid * n + sid * rpt, rpt)])

    return agg


def _make_cnt(n, e):
    nw = NC * NS
    epw = e // nw
    groups = epw // (G * K)
    nf = 10
    rpt = n // nf
    mesh = plsc.VectorSubcoreMesh(core_axis_name="c", subcore_axis_name="s")

    @functools.partial(
        pl.kernel,
        out_type=jax.ShapeDtypeStruct((NC * n, CW), jnp.float32),
        mesh=mesh,
        scratch_types=[
            pltpu.VMEM_SHARED((n, CW), jnp.float32),
            [pltpu.VMEM((K,), jnp.int32) for _ in range(G)],
            pltpu.VMEM((K, CW), jnp.float32),
        ],
    )
    def cnt(dst_hbm, ones_hbm, zeros_hbm, out_hbm, acc_sh, didx, ones_v):
        cid = lax.axis_index("c")
        sid = lax.axis_index("s")
        wid = sid * NC + cid

        @pl.when(sid < nf)
        def _():
            pltpu.sync_copy(zeros_hbm, acc_sh.at[pl.ds(sid * rpt, rpt)])
        pltpu.sync_copy(ones_hbm, ones_v)
        plsc.subcore_barrier()

        base = wid * epw

        def group_body(g, _):
            off = base + g * (G * K)
            for b in range(G):
                pltpu.sync_copy(dst_hbm.at[pl.ds(off + b * K, K)], didx[b])
            for b in range(G):
                pltpu.sync_copy(ones_v, acc_sh.at[didx[b]], add=True)
            return 0

        lax.fori_loop(0, groups, group_body, 0)

        plsc.subcore_barrier()

        @pl.when(sid < nf)
        def _():
            pltpu.sync_copy(acc_sh.at[pl.ds(sid * rpt, rpt)],
                            out_hbm.at[pl.ds(cid * n + sid * rpt, rpt)])

    return cnt


def _dot(a, b):
    return jax.lax.dot_general(a, b, (((1,), (0,)), ((), ())),
                               preferred_element_type=jnp.float32)


def _pre_body(x_ref, win_ref, bin_ref, w1_ref, b1_ref, w2_ref, b2_ref,
              bv_ref, cntp_ref,
              z_ref, msg_ref, acc_ref, inv_ref, betas_ref):
    z = _dot(x_ref[...], win_ref[...]) + bin_ref[...]
    bv = bv_ref[...]
    ex = jnp.exp(bv - jnp.max(bv))
    betas = ex / jnp.sum(ex)
    betas_ref[...] = betas
    cnt = 1.0 + jnp.sum(cntp_ref[...][:, :, 0], axis=0)
    inv_ref[...] = (1.0 / cnt)[:, None]
    z_ref[...] = z
    acc_ref[...] = betas[0, 0] * z
    h = jnp.maximum(_dot(z, w1_ref[...]) + b1_ref[...], 0.0)
    msg_ref[...] = _dot(h, w2_ref[...]) + b2_ref[...]


def _layer_body(lidx, has_next,
                z_ref, s_ref, msg_ref, inv_ref, acc_ref, betas_ref,
                u1_ref, u1b_ref, u2_ref, u2b_ref, g_ref, bln_ref,
                *rest):
    if has_next:
        (w1_ref, b1_ref, w2_ref, b2_ref,
         zo_ref, acco_ref, msgo_ref) = rest
    else:
        zo_ref, acco_ref = rest
    sblk = s_ref[...]
    s = z_ref[...] + (sblk[0] + sblk[1] + msg_ref[...]) * inv_ref[...]
    h2 = jnp.maximum(_dot(s, u1_ref[...]) + u1b_ref[...], 0.0)
    o = _dot(h2, u2_ref[...]) + u2b_ref[...]
    mu = jnp.mean(o, axis=-1, keepdims=True)
    var = jnp.mean((o - mu) ** 2, axis=-1, keepdims=True)
    zn = (o - mu) * lax.rsqrt(var + 1e-5) * g_ref[...] + bln_ref[...]
    zo_ref[...] = zn
    acco_ref[...] = acc_ref[...] + betas_ref[0, lidx + 1] * zn
    if has_next:
        h = jnp.maximum(_dot(zn, w1_ref[...]) + b1_ref[...], 0.0)
        msgo_ref[...] = _dot(h, w2_ref[...]) + b2_ref[...]


def _full(shape):
    nd = len(shape)
    return pl.BlockSpec(shape, lambda i: (0,) * nd)


def _make_pre(n, d, nl, bn):
    grid = (n // bn,)
    row = pl.BlockSpec((bn, d), lambda i: (i, 0))
    return pl.pallas_call(
        _pre_body,
        grid=grid,
        in_specs=[
            row, _full((d, d)), _full((1, d)),
            _full((d, d)), _full((1, d)), _full((d, d)), _full((1, d)),
            _full((1, nl + 1)),
            pl.BlockSpec((NC, bn, CW), lambda i: (0, i, 0)),
        ],
        out_specs=[row, row, row,
                   pl.BlockSpec((bn, 1), lambda i: (i, 0)),
                   _full((1, nl + 1))],
        out_shape=[
            jax.ShapeDtypeStruct((n, d), jnp.float32),
            jax.ShapeDtypeStruct((n, d), jnp.float32),
            jax.ShapeDtypeStruct((n, d), jnp.float32),
            jax.ShapeDtypeStruct((n, 1), jnp.float32),
            jax.ShapeDtypeStruct((1, nl + 1), jnp.float32),
        ],
    )


def _make_layer(n, d, nl, bn, lidx, has_next):
    grid = (n // bn,)
    row = pl.BlockSpec((bn, d), lambda i: (i, 0))
    in_specs = [
        row,
        pl.BlockSpec((NC, bn, d), lambda i: (0, i, 0)),
        row,
        pl.BlockSpec((bn, 1), lambda i: (i, 0)),
        row,
        _full((1, nl + 1)),
        _full((d, d)), _full((1, d)), _full((d, d)), _full((1, d)),
        _full((1, d)), _full((1, d)),
    ]
    out_specs = [row, row]
    out_shape = [
        jax.ShapeDtypeStruct((n, d), jnp.float32),
        jax.ShapeDtypeStruct((n, d), jnp.float32),
    ]
    if has_next:
        in_specs += [_full((d, d)), _full((1, d)), _full((d, d)), _full((1, d))]
        out_specs.append(row)
        out_shape.append(jax.ShapeDtypeStruct((n, d), jnp.float32))
    return pl.pallas_call(
        functools.partial(_layer_body, lidx, has_next),
        grid=grid,
        in_specs=in_specs,
        out_specs=out_specs,
        out_shape=out_shape,
    )


def kernel(x, edge_index, W_in, b_in, W1, b1, W2, b2, U1, u1, U2, u2,
           gamma, beta_ln, beta_vec):
    n, d = x.shape
    e = edge_index.shape[1]
    nl = W1.shape[0]
    bn = 2000
    rpt = n // 10

    src1 = edge_index[0]
    dst1 = edge_index[1]
    cpw = _agg_chunks(e)
    ep = NC * NS * cpw * KA
    srcp = jnp.concatenate(
        [src1, jnp.zeros((ep - e,), jnp.int32)]).reshape(-1, KA)
    dstp = jnp.concatenate(
        [dst1, jnp.full((ep - e,), n, jnp.int32)]).reshape(-1, KA)
    zeros_d = jnp.zeros((rpt, d), jnp.float32)
    zeros_c = jnp.zeros((rpt, CW), jnp.float32)
    ones_c = jnp.ones((K, CW), jnp.float32)

    cnt_k = _make_cnt(n, e)
    agg_k = _make_agg(n, e, d)
    pre_k = _make_pre(n, d, nl, bn)

    cntp = cnt_k(dst1, ones_c, zeros_c).reshape(NC, n, CW)
    z, msg, acc, inv, betas = pre_k(
        x, W_in, b_in.reshape(1, d),
        W1[0], b1[0].reshape(1, d), W2[0], b2[0].reshape(1, d),
        beta_vec.reshape(1, nl + 1), cntp)

    for l in range(nl):
        S = agg_k(msg, srcp, dstp, zeros_d).reshape(NC, n, d)
        args = [z, S, msg, inv, acc, betas,
                U1[l], u1[l].reshape(1, d), U2[l], u2[l].reshape(1, d),
                gamma[l].reshape(1, d), beta_ln[l].reshape(1, d)]
        if l + 1 < nl:
            args += [W1[l + 1], b1[l + 1].reshape(1, d),
                     W2[l + 1], b2[l + 1].reshape(1, d)]
            z, acc, msg = _make_layer(n, d, nl, bn, l, True)(*args)
        else:
            z, acc = _make_layer(n, d, nl, bn, l, False)(*args)
    return acc


# speedup vs baseline: 1.9903x; 1.9903x over previous
import functools

import jax
import jax.numpy as jnp
from jax import lax
from jax.experimental import pallas as pl
from jax.experimental.pallas import tpu as pltpu
from jax.experimental.pallas import tpu_sc as plsc

NC = 2
NS = 16
K = 40
G = 5
KA = 40
NBUF = 4
CW = 128


PH = 4


def _agg_chunks(e):
    nw = NC * NS
    cpw = -(-e // (nw * KA))
    return (cpw + 8 * PH - 1) // (8 * PH) * (8 * PH)


def _make_agg(n, e, d):
    cpw = _agg_chunks(e)
    cpp = cpw // PH
    nf = 10
    rpt = n // nf
    mesh = plsc.VectorSubcoreMesh(core_axis_name="c", subcore_axis_name="s")

    @functools.partial(
        pl.kernel,
        out_type=jax.ShapeDtypeStruct((NC * n, d), jnp.float32),
        mesh=mesh,
        scratch_types=[
            pltpu.VMEM_SHARED((n + 8, d), jnp.float32),
            pltpu.VMEM((cpp, KA), jnp.int32),
            pltpu.VMEM((cpp, KA), jnp.int32),
            pltpu.VMEM((NBUF, KA, d), jnp.float32),
            pltpu.SemaphoreType.DMA,
        ],
    )
    def agg(msg_hbm, srcp_hbm, dstp_hbm, zeros_hbm, out_hbm,
            acc_sh, sidx, didx, rows, semg):
        cid = lax.axis_index("c")
        sid = lax.axis_index("s")
        wid = sid * NC + cid
        @pl.when(sid < nf)
        def _():
            pltpu.sync_copy(zeros_hbm, acc_sh.at[pl.ds(sid * rpt, rpt)])
        plsc.subcore_barrier()

        for p in range(PH):
            row0 = wid * cpw + p * cpp
            pltpu.sync_copy(srcp_hbm.at[pl.ds(row0, cpp)], sidx)
            pltpu.sync_copy(dstp_hbm.at[pl.ds(row0, cpp)], didx)

            def group(gi, _):
                c0 = gi * NBUF
                cps = [
                    pltpu.async_copy(msg_hbm.at[sidx.at[c0 + b]],
                                     rows.at[b], semg)
                    for b in range(NBUF)
                ]
                for b in range(NBUF):
                    cps[b].wait()
                for b in range(NBUF):
                    pltpu.sync_copy(rows.at[b], acc_sh.at[didx.at[c0 + b]],
                                    add=True)
                return 0

            lax.fori_loop(0, cpp // NBUF, group, 0)

        plsc.subcore_barrier()

        @pl.when(sid < nf)
        def _():
            pltpu.sync_copy(acc_sh.at[pl.ds(sid * rpt, rpt)],
                            out_hbm.at[pl.ds(cid * n + sid * rpt, rpt)])

    return agg


def _make_cnt(n, e):
    nw = NC * NS
    epw = e // nw
    groups = epw // (G * K)
    nf = 10
    rpt = n // nf
    mesh = plsc.VectorSubcoreMesh(core_axis_name="c", subcore_axis_name="s")

    @functools.partial(
        pl.kernel,
        out_type=jax.ShapeDtypeStruct((NC * n, CW), jnp.float32),
        mesh=mesh,
        scratch_types=[
            pltpu.VMEM_SHARED((n, CW), jnp.float32),
            [pltpu.VMEM((K,), jnp.int32) for _ in range(G)],
            pltpu.VMEM((K, CW), jnp.float32),
        ],
    )
    def cnt(dst_hbm, ones_hbm, zeros_hbm, out_hbm, acc_sh, didx, ones_v):
        cid = lax.axis_index("c")
        sid = lax.axis_index("s")
        wid = sid * NC + cid

        @pl.when(sid < nf)
        def _():
            pltpu.sync_copy(zeros_hbm, acc_sh.at[pl.ds(sid * rpt, rpt)])
        pltpu.sync_copy(ones_hbm, ones_v)
        plsc.subcore_barrier()

        base = wid * epw

        def group_body(g, _):
            off = base + g * (G * K)
            for b in range(G):
                pltpu.sync_copy(dst_hbm.at[pl.ds(off + b * K, K)], didx[b])
            for b in range(G):
                pltpu.sync_copy(ones_v, acc_sh.at[didx[b]], add=True)
            return 0

        lax.fori_loop(0, groups, group_body, 0)

        plsc.subcore_barrier()

        @pl.when(sid < nf)
        def _():
            pltpu.sync_copy(acc_sh.at[pl.ds(sid * rpt, rpt)],
                            out_hbm.at[pl.ds(cid * n + sid * rpt, rpt)])

    return cnt


def _dot(a, b):
    return jax.lax.dot_general(a, b, (((1,), (0,)), ((), ())),
                               preferred_element_type=jnp.float32)


def _pre_body(x_ref, win_ref, bin_ref, w1_ref, b1_ref, w2_ref, b2_ref,
              bv_ref, cntp_ref,
              z_ref, msg_ref, acc_ref, inv_ref, betas_ref):
    z = _dot(x_ref[...], win_ref[...]) + bin_ref[...]
    bv = bv_ref[...]
    ex = jnp.exp(bv - jnp.max(bv))
    betas = ex / jnp.sum(ex)
    betas_ref[...] = betas
    cnt = 1.0 + jnp.sum(cntp_ref[...][:, :, 0], axis=0)
    inv_ref[...] = (1.0 / cnt)[:, None]
    z_ref[...] = z
    acc_ref[...] = betas[0, 0] * z
    h = jnp.maximum(_dot(z, w1_ref[...]) + b1_ref[...], 0.0)
    msg_ref[...] = _dot(h, w2_ref[...]) + b2_ref[...]


def _layer_body(lidx, has_next,
                z_ref, s_ref, msg_ref, inv_ref, acc_ref, betas_ref,
                u1_ref, u1b_ref, u2_ref, u2b_ref, g_ref, bln_ref,
                *rest):
    if has_next:
        (w1_ref, b1_ref, w2_ref, b2_ref,
         zo_ref, acco_ref, msgo_ref) = rest
    else:
        zo_ref, acco_ref = rest
    sblk = s_ref[...]
    s = z_ref[...] + (sblk[0] + sblk[1] + msg_ref[...]) * inv_ref[...]
    h2 = jnp.maximum(_dot(s, u1_ref[...]) + u1b_ref[...], 0.0)
    o = _dot(h2, u2_ref[...]) + u2b_ref[...]
    mu = jnp.mean(o, axis=-1, keepdims=True)
    var = jnp.mean((o - mu) ** 2, axis=-1, keepdims=True)
    zn = (o - mu) * lax.rsqrt(var + 1e-5) * g_ref[...] + bln_ref[...]
    zo_ref[...] = zn
    acco_ref[...] = acc_ref[...] + betas_ref[0, lidx + 1] * zn
    if has_next:
        h = jnp.maximum(_dot(zn, w1_ref[...]) + b1_ref[...], 0.0)
        msgo_ref[...] = _dot(h, w2_ref[...]) + b2_ref[...]


def _full(shape):
    nd = len(shape)
    return pl.BlockSpec(shape, lambda i: (0,) * nd)


def _make_pre(n, d, nl, bn):
    grid = (n // bn,)
    row = pl.BlockSpec((bn, d), lambda i: (i, 0))
    return pl.pallas_call(
        _pre_body,
        grid=grid,
        in_specs=[
            row, _full((d, d)), _full((1, d)),
            _full((d, d)), _full((1, d)), _full((d, d)), _full((1, d)),
            _full((1, nl + 1)),
            pl.BlockSpec((NC, bn, CW), lambda i: (0, i, 0)),
        ],
        out_specs=[row, row, row,
                   pl.BlockSpec((bn, 1), lambda i: (i, 0)),
                   _full((1, nl + 1))],
        out_shape=[
            jax.ShapeDtypeStruct((n, d), jnp.float32),
            jax.ShapeDtypeStruct((n, d), jnp.float32),
            jax.ShapeDtypeStruct((n, d), jnp.float32),
            jax.ShapeDtypeStruct((n, 1), jnp.float32),
            jax.ShapeDtypeStruct((1, nl + 1), jnp.float32),
        ],
    )


def _make_layer(n, d, nl, bn, lidx, has_next):
    grid = (n // bn,)
    row = pl.BlockSpec((bn, d), lambda i: (i, 0))
    in_specs = [
        row,
        pl.BlockSpec((NC, bn, d), lambda i: (0, i, 0)),
        row,
        pl.BlockSpec((bn, 1), lambda i: (i, 0)),
        row,
        _full((1, nl + 1)),
        _full((d, d)), _full((1, d)), _full((d, d)), _full((1, d)),
        _full((1, d)), _full((1, d)),
    ]
    out_specs = [row, row]
    out_shape = [
        jax.ShapeDtypeStruct((n, d), jnp.float32),
        jax.ShapeDtypeStruct((n, d), jnp.float32),
    ]
    if has_next:
        in_specs += [_full((d, d)), _full((1, d)), _full((d, d)), _full((1, d))]
        out_specs.append(row)
        out_shape.append(jax.ShapeDtypeStruct((n, d), jnp.float32))
    return pl.pallas_call(
        functools.partial(_layer_body, lidx, has_next),
        grid=grid,
        in_specs=in_specs,
        out_specs=out_specs,
        out_shape=out_shape,
    )


def kernel(x, edge_index, W_in, b_in, W1, b1, W2, b2, U1, u1, U2, u2,
           gamma, beta_ln, beta_vec):
    n, d = x.shape
    e = edge_index.shape[1]
    nl = W1.shape[0]
    bn = 2000
    rpt = n // 10

    src1 = edge_index[0]
    dst1 = edge_index[1]
    cpw = _agg_chunks(e)
    ep = NC * NS * cpw * KA
    srcp = jnp.concatenate(
        [src1, jnp.zeros((ep - e,), jnp.int32)]).reshape(-1, KA)
    dstp = jnp.concatenate(
        [dst1, jnp.full((ep - e,), n, jnp.int32)]).reshape(-1, KA)
    zeros_d = jnp.zeros((rpt, d), jnp.float32)
    zeros_c = jnp.zeros((rpt, CW), jnp.float32)
    ones_c = jnp.ones((K, CW), jnp.float32)

    cnt_k = _make_cnt(n, e)
    agg_k = _make_agg(n, e, d)
    pre_k = _make_pre(n, d, nl, bn)

    cntp = cnt_k(dst1, ones_c, zeros_c).reshape(NC, n, CW)
    z, msg, acc, inv, betas = pre_k(
        x, W_in, b_in.reshape(1, d),
        W1[0], b1[0].reshape(1, d), W2[0], b2[0].reshape(1, d),
        beta_vec.reshape(1, nl + 1), cntp)

    for l in range(nl):
        S = agg_k(msg, srcp, dstp, zeros_d).reshape(NC, n, d)
        args = [z, S, msg, inv, acc, betas,
                U1[l], u1[l].reshape(1, d), U2[l], u2[l].reshape(1, d),
                gamma[l].reshape(1, d), beta_ln[l].reshape(1, d)]
        if l + 1 < nl:
            args += [W1[l + 1], b1[l + 1].reshape(1, d),
                     W2[l + 1], b2[l + 1].reshape(1, d)]
            z, acc, msg = _make_layer(n, d, nl, bn, l, True)(*args)
        else:
            z, acc = _make_layer(n, d, nl, bn, l, False)(*args)
    return acc


# speedup vs baseline: 3.2534x; 1.6346x over previous
import functools

import jax
import jax.numpy as jnp
from jax import lax
from jax.experimental import pallas as pl
from jax.experimental.pallas import tpu as pltpu
from jax.experimental.pallas import tpu_sc as plsc

NC = 2
NS = 16
K = 40
G = 5
KA = 40
CW = 128


def _make_agg(n, e, d):
    nw = NC * NS
    epw = e // nw
    groups = epw // (G * KA)
    nf = 10
    rpt = n // nf
    mesh = plsc.VectorSubcoreMesh(core_axis_name="c", subcore_axis_name="s")

    @functools.partial(
        pl.kernel,
        out_type=jax.ShapeDtypeStruct((NC * n, d), jnp.float32),
        mesh=mesh,
        scratch_types=[
            pltpu.VMEM_SHARED((n, d), jnp.float32),
            pltpu.VMEM((G * KA,), jnp.int32),
            [pltpu.VMEM((KA,), jnp.int32) for _ in range(G)],
            pltpu.VMEM((G, KA, d), jnp.float32),
            pltpu.SemaphoreType.DMA,
        ],
    )
    def agg(msg_hbm, src_hbm, dst_hbm, zeros_hbm, out_hbm,
            acc_sh, sidx, didx, rows, sem):
        cid = lax.axis_index("c")
        sid = lax.axis_index("s")
        wid = sid * NC + cid
        @pl.when(sid < nf)
        def _():
            pltpu.sync_copy(zeros_hbm, acc_sh.at[pl.ds(sid * rpt, rpt)])
        plsc.subcore_barrier()

        base = wid * epw

        def group_body(g, _):
            off = base + g * (G * KA)
            pltpu.sync_copy(src_hbm.at[pl.ds(off, G * KA)], sidx)
            for b in range(G):
                pltpu.sync_copy(dst_hbm.at[pl.ds(off + b * KA, KA)], didx[b])
            cps = [
                pltpu.async_copy(msg_hbm.at[sidx.at[pl.ds(b * KA, KA)]],
                                 rows.at[b], sem)
                for b in range(G)
            ]
            for b in range(G):
                cps[b].wait()
            for b in range(G):
                pltpu.sync_copy(rows.at[b], acc_sh.at[didx[b]], add=True)
            return 0

        lax.fori_loop(0, groups, group_body, 0)

        plsc.subcore_barrier()

        @pl.when(sid < nf)
        def _():
            pltpu.sync_copy(acc_sh.at[pl.ds(sid * rpt, rpt)],
                            out_hbm.at[pl.ds(cid * n + sid * rpt, rpt)])

    return agg


def _make_cnt(n, e):
    nw = NC * NS
    epw = e // nw
    groups = epw // (G * K)
    nf = 10
    rpt = n // nf
    mesh = plsc.VectorSubcoreMesh(core_axis_name="c", subcore_axis_name="s")

    @functools.partial(
        pl.kernel,
        out_type=jax.ShapeDtypeStruct((NC * n, CW), jnp.float32),
        mesh=mesh,
        scratch_types=[
            pltpu.VMEM_SHARED((n, CW), jnp.float32),
            [pltpu.VMEM((K,), jnp.int32) for _ in range(G)],
            pltpu.VMEM((K, CW), jnp.float32),
        ],
    )
    def cnt(dst_hbm, ones_hbm, zeros_hbm, out_hbm, acc_sh, didx, ones_v):
        cid = lax.axis_index("c")
        sid = lax.axis_index("s")
        wid = sid * NC + cid

        @pl.when(sid < nf)
        def _():
            pltpu.sync_copy(zeros_hbm, acc_sh.at[pl.ds(sid * rpt, rpt)])
        pltpu.sync_copy(ones_hbm, ones_v)
        plsc.subcore_barrier()

        base = wid * epw

        def group_body(g, _):
            off = base + g * (G * K)
            for b in range(G):
                pltpu.sync_copy(dst_hbm.at[pl.ds(off + b * K, K)], didx[b])
            for b in range(G):
                pltpu.sync_copy(ones_v, acc_sh.at[didx[b]], add=True)
            return 0

        lax.fori_loop(0, groups, group_body, 0)

        plsc.subcore_barrier()

        @pl.when(sid < nf)
        def _():
            pltpu.sync_copy(acc_sh.at[pl.ds(sid * rpt, rpt)],
                            out_hbm.at[pl.ds(cid * n + sid * rpt, rpt)])

    return cnt


def _dot(a, b):
    return jax.lax.dot_general(a, b, (((1,), (0,)), ((), ())),
                               preferred_element_type=jnp.float32)


def _pre_body(x_ref, win_ref, bin_ref, w1_ref, b1_ref, w2_ref, b2_ref,
              bv_ref, cntp_ref,
              z_ref, msg_ref, acc_ref, inv_ref, betas_ref):
    z = _dot(x_ref[...], win_ref[...]) + bin_ref[...]
    bv = bv_ref[...]
    ex = jnp.exp(bv - jnp.max(bv))
    betas = ex / jnp.sum(ex)
    betas_ref[...] = betas
    cnt = 1.0 + jnp.sum(cntp_ref[...][:, :, 0], axis=0)
    inv_ref[...] = (1.0 / cnt)[:, None]
    z_ref[...] = z
    acc_ref[...] = betas[0, 0] * z
    h = jnp.maximum(_dot(z, w1_ref[...]) + b1_ref[...], 0.0)
    msg_ref[...] = _dot(h, w2_ref[...]) + b2_ref[...]


def _layer_body(lidx, has_next,
                z_ref, s_ref, msg_ref, inv_ref, acc_ref, betas_ref,
                u1_ref, u1b_ref, u2_ref, u2b_ref, g_ref, bln_ref,
                *rest):
    if has_next:
        (w1_ref, b1_ref, w2_ref, b2_ref,
         zo_ref, acco_ref, msgo_ref) = rest
    else:
        zo_ref, acco_ref = rest
    sblk = s_ref[...]
    s = z_ref[...] + (sblk[0] + sblk[1] + msg_ref[...]) * inv_ref[...]
    h2 = jnp.maximum(_dot(s, u1_ref[...]) + u1b_ref[...], 0.0)
    o = _dot(h2, u2_ref[...]) + u2b_ref[...]
    mu = jnp.mean(o, axis=-1, keepdims=True)
    var = jnp.mean((o - mu) ** 2, axis=-1, keepdims=True)
    zn = (o - mu) * lax.rsqrt(var + 1e-5) * g_ref[...] + bln_ref[...]
    zo_ref[...] = zn
    acco_ref[...] = acc_ref[...] + betas_ref[0, lidx + 1] * zn
    if has_next:
        h = jnp.maximum(_dot(zn, w1_ref[...]) + b1_ref[...], 0.0)
        msgo_ref[...] = _dot(h, w2_ref[...]) + b2_ref[...]


def _full(shape):
    nd = len(shape)
    return pl.BlockSpec(shape, lambda i: (0,) * nd)


def _make_pre(n, d, nl, bn):
    grid = (n // bn,)
    row = pl.BlockSpec((bn, d), lambda i: (i, 0))
    return pl.pallas_call(
        _pre_body,
        grid=grid,
        in_specs=[
            row, _full((d, d)), _full((1, d)),
            _full((d, d)), _full((1, d)), _full((d, d)), _full((1, d)),
            _full((1, nl + 1)),
            pl.BlockSpec((NC, bn, CW), lambda i: (0, i, 0)),
        ],
        out_specs=[row, row, row,
                   pl.BlockSpec((bn, 1), lambda i: (i, 0)),
                   _full((1, nl + 1))],
        out_shape=[
            jax.ShapeDtypeStruct((n, d), jnp.float32),
            jax.ShapeDtypeStruct((n, d), jnp.float32),
            jax.ShapeDtypeStruct((n, d), jnp.float32),
            jax.ShapeDtypeStruct((n, 1), jnp.float32),
            jax.ShapeDtypeStruct((1, nl + 1), jnp.float32),
        ],
    )


def _make_layer(n, d, nl, bn, lidx, has_next):
    grid = (n // bn,)
    row = pl.BlockSpec((bn, d), lambda i: (i, 0))
    in_specs = [
        row,
        pl.BlockSpec((NC, bn, d), lambda i: (0, i, 0)),
        row,
        pl.BlockSpec((bn, 1), lambda i: (i, 0)),
        row,
        _full((1, nl + 1)),
        _full((d, d)), _full((1, d)), _full((d, d)), _full((1, d)),
        _full((1, d)), _full((1, d)),
    ]
    out_specs = [row, row]
    out_shape = [
        jax.ShapeDtypeStruct((n, d), jnp.float32),
        jax.ShapeDtypeStruct((n, d), jnp.float32),
    ]
    if has_next:
        in_specs += [_full((d, d)), _full((1, d)), _full((d, d)), _full((1, d))]
        out_specs.append(row)
        out_shape.append(jax.ShapeDtypeStruct((n, d), jnp.float32))
    return pl.pallas_call(
        functools.partial(_layer_body, lidx, has_next),
        grid=grid,
        in_specs=in_specs,
        out_specs=out_specs,
        out_shape=out_shape,
    )


def kernel(x, edge_index, W_in, b_in, W1, b1, W2, b2, U1, u1, U2, u2,
           gamma, beta_ln, beta_vec):
    n, d = x.shape
    e = edge_index.shape[1]
    nl = W1.shape[0]
    bn = 2000
    rpt = n // 10

    src1 = edge_index[0]
    dst1 = edge_index[1]
    zeros_d = jnp.zeros((rpt, d), jnp.float32)
    zeros_c = jnp.zeros((rpt, CW), jnp.float32)
    ones_c = jnp.ones((K, CW), jnp.float32)

    cnt_k = _make_cnt(n, e)
    agg_k = _make_agg(n, e, d)
    pre_k = _make_pre(n, d, nl, bn)

    cntp = cnt_k(dst1, ones_c, zeros_c).reshape(NC, n, CW)
    z, msg, acc, inv, betas = pre_k(
        x, W_in, b_in.reshape(1, d),
        W1[0], b1[0].reshape(1, d), W2[0], b2[0].reshape(1, d),
        beta_vec.reshape(1, nl + 1), cntp)

    for l in range(nl):
        S = agg_k(msg, src1, dst1, zeros_d).reshape(NC, n, d)
        args = [z, S, msg, inv, acc, betas,
                U1[l], u1[l].reshape(1, d), U2[l], u2[l].reshape(1, d),
                gamma[l].reshape(1, d), beta_ln[l].reshape(1, d)]
        if l + 1 < nl:
            args += [W1[l + 1], b1[l + 1].reshape(1, d),
                     W2[l + 1], b2[l + 1].reshape(1, d)]
            z, acc, msg = _make_layer(n, d, nl, bn, l, True)(*args)
        else:
            z, acc = _make_layer(n, d, nl, bn, l, False)(*args)
    return acc


# speedup vs baseline: 3.6511x; 1.1223x over previous
import functools

import jax
import jax.numpy as jnp
from jax import lax
from jax.experimental import pallas as pl
from jax.experimental.pallas import tpu as pltpu
from jax.experimental.pallas import tpu_sc as plsc

NC = 2
NS = 16
K = 40
G = 5
KA = 40
CW = 128


def _make_agg(n, e, d):
    nw = NC * NS
    epw = e // nw
    groups = epw // (G * KA)
    nf = 10
    rpt = n // nf
    mesh = plsc.VectorSubcoreMesh(core_axis_name="c", subcore_axis_name="s")

    @functools.partial(
        pl.kernel,
        out_type=jax.ShapeDtypeStruct((NC * n, d), jnp.float32),
        mesh=mesh,
        scratch_types=[
            pltpu.VMEM_SHARED((n, d), jnp.float32),
            pltpu.VMEM((G * KA,), jnp.int32),
            [pltpu.VMEM((KA,), jnp.int32) for _ in range(G)],
            pltpu.VMEM((G, KA, d), jnp.float32),
            pltpu.SemaphoreType.DMA,
        ],
    )
    def agg(msg_hbm, src_hbm, dst_hbm, zeros_hbm, out_hbm,
            acc_sh, sidx, didx, rows, sem):
        cid = lax.axis_index("c")
        sid = lax.axis_index("s")
        wid = sid * NC + cid
        @pl.when(sid < nf)
        def _():
            pltpu.sync_copy(zeros_hbm, acc_sh.at[pl.ds(sid * rpt, rpt)])
        plsc.subcore_barrier()

        base = wid * epw

        def group_body(g, _):
            off = base + g * (G * KA)
            pltpu.sync_copy(src_hbm.at[pl.ds(off, G * KA)], sidx)
            for b in range(G):
                pltpu.sync_copy(dst_hbm.at[pl.ds(off + b * KA, KA)], didx[b])
            cps = [
                pltpu.async_copy(msg_hbm.at[sidx.at[pl.ds(b * KA, KA)]],
                                 rows.at[b], sem)
                for b in range(G)
            ]
            for b in range(G):
                cps[b].wait()
                pltpu.sync_copy(rows.at[b], acc_sh.at[didx[b]], add=True)
            return 0

        lax.fori_loop(0, groups, group_body, 0)

        plsc.subcore_barrier()

        @pl.when(sid < nf)
        def _():
            pltpu.sync_copy(acc_sh.at[pl.ds(sid * rpt, rpt)],
                            out_hbm.at[pl.ds(cid * n + sid * rpt, rpt)])

    return agg


def _make_cnt(n, e):
    nw = NC * NS
    epw = e // nw
    groups = epw // (G * K)
    nf = 10
    rpt = n // nf
    mesh = plsc.VectorSubcoreMesh(core_axis_name="c", subcore_axis_name="s")

    @functools.partial(
        pl.kernel,
        out_type=jax.ShapeDtypeStruct((NC * n, CW), jnp.float32),
        mesh=mesh,
        scratch_types=[
            pltpu.VMEM_SHARED((n, CW), jnp.float32),
            [pltpu.VMEM((K,), jnp.int32) for _ in range(G)],
            pltpu.VMEM((K, CW), jnp.float32),
        ],
    )
    def cnt(dst_hbm, ones_hbm, zeros_hbm, out_hbm, acc_sh, didx, ones_v):
        cid = lax.axis_index("c")
        sid = lax.axis_index("s")
        wid = sid * NC + cid

        @pl.when(sid < nf)
        def _():
            pltpu.sync_copy(zeros_hbm, acc_sh.at[pl.ds(sid * rpt, rpt)])
        pltpu.sync_copy(ones_hbm, ones_v)
        plsc.subcore_barrier()

        base = wid * epw

        def group_body(g, _):
            off = base + g * (G * K)
            for b in range(G):
                pltpu.sync_copy(dst_hbm.at[pl.ds(off + b * K, K)], didx[b])
            for b in range(G):
                pltpu.sync_copy(ones_v, acc_sh.at[didx[b]], add=True)
            return 0

        lax.fori_loop(0, groups, group_body, 0)

        plsc.subcore_barrier()

        @pl.when(sid < nf)
        def _():
            pltpu.sync_copy(acc_sh.at[pl.ds(sid * rpt, rpt)],
                            out_hbm.at[pl.ds(cid * n + sid * rpt, rpt)])

    return cnt


def _dot(a, b):
    return jax.lax.dot_general(a, b, (((1,), (0,)), ((), ())),
                               preferred_element_type=jnp.float32)


def _pre_body(x_ref, win_ref, bin_ref, w1_ref, b1_ref, w2_ref, b2_ref,
              bv_ref, cntp_ref,
              z_ref, msg_ref, acc_ref, inv_ref, betas_ref):
    z = _dot(x_ref[...], win_ref[...]) + bin_ref[...]
    bv = bv_ref[...]
    ex = jnp.exp(bv - jnp.max(bv))
    betas = ex / jnp.sum(ex)
    betas_ref[...] = betas
    cnt = 1.0 + jnp.sum(cntp_ref[...][:, :, 0], axis=0)
    inv_ref[...] = (1.0 / cnt)[:, None]
    z_ref[...] = z
    acc_ref[...] = betas[0, 0] * z
    h = jnp.maximum(_dot(z, w1_ref[...]) + b1_ref[...], 0.0)
    msg_ref[...] = _dot(h, w2_ref[...]) + b2_ref[...]


def _layer_body(lidx, has_next,
                z_ref, s_ref, msg_ref, inv_ref, acc_ref, betas_ref,
                u1_ref, u1b_ref, u2_ref, u2b_ref, g_ref, bln_ref,
                *rest):
    if has_next:
        (w1_ref, b1_ref, w2_ref, b2_ref,
         zo_ref, acco_ref, msgo_ref) = rest
    else:
        zo_ref, acco_ref = rest
    sblk = s_ref[...]
    s = z_ref[...] + (sblk[0] + sblk[1] + msg_ref[...]) * inv_ref[...]
    h2 = jnp.maximum(_dot(s, u1_ref[...]) + u1b_ref[...], 0.0)
    o = _dot(h2, u2_ref[...]) + u2b_ref[...]
    mu = jnp.mean(o, axis=-1, keepdims=True)
    var = jnp.mean((o - mu) ** 2, axis=-1, keepdims=True)
    zn = (o - mu) * lax.rsqrt(var + 1e-5) * g_ref[...] + bln_ref[...]
    zo_ref[...] = zn
    acco_ref[...] = acc_ref[...] + betas_ref[0, lidx + 1] * zn
    if has_next:
        h = jnp.maximum(_dot(zn, w1_ref[...]) + b1_ref[...], 0.0)
        msgo_ref[...] = _dot(h, w2_ref[...]) + b2_ref[...]


def _full(shape):
    nd = len(shape)
    return pl.BlockSpec(shape, lambda i: (0,) * nd)


def _make_pre(n, d, nl, bn):
    grid = (n // bn,)
    row = pl.BlockSpec((bn, d), lambda i: (i, 0))
    return pl.pallas_call(
        _pre_body,
        grid=grid,
        in_specs=[
            row, _full((d, d)), _full((1, d)),
            _full((d, d)), _full((1, d)), _full((d, d)), _full((1, d)),
            _full((1, nl + 1)),
            pl.BlockSpec((NC, bn, CW), lambda i: (0, i, 0)),
        ],
        out_specs=[row, row, row,
                   pl.BlockSpec((bn, 1), lambda i: (i, 0)),
                   _full((1, nl + 1))],
        out_shape=[
            jax.ShapeDtypeStruct((n, d), jnp.float32),
            jax.ShapeDtypeStruct((n, d), jnp.float32),
            jax.ShapeDtypeStruct((n, d), jnp.float32),
            jax.ShapeDtypeStruct((n, 1), jnp.float32),
            jax.ShapeDtypeStruct((1, nl + 1), jnp.float32),
        ],
    )


def _make_layer(n, d, nl, bn, lidx, has_next):
    grid = (n // bn,)
    row = pl.BlockSpec((bn, d), lambda i: (i, 0))
    in_specs = [
        row,
        pl.BlockSpec((NC, bn, d), lambda i: (0, i, 0)),
        row,
        pl.BlockSpec((bn, 1), lambda i: (i, 0)),
        row,
        _full((1, nl + 1)),
        _full((d, d)), _full((1, d)), _full((d, d)), _full((1, d)),
        _full((1, d)), _full((1, d)),
    ]
    out_specs = [row, row]
    out_shape = [
        jax.ShapeDtypeStruct((n, d), jnp.float32),
        jax.ShapeDtypeStruct((n, d), jnp.float32),
    ]
    if has_next:
        in_specs += [_full((d, d)), _full((1, d)), _full((d, d)), _full((1, d))]
        out_specs.append(row)
        out_shape.append(jax.ShapeDtypeStruct((n, d), jnp.float32))
    return pl.pallas_call(
        functools.partial(_layer_body, lidx, has_next),
        grid=grid,
        in_specs=in_specs,
        out_specs=out_specs,
        out_shape=out_shape,
    )


def kernel(x, edge_index, W_in, b_in, W1, b1, W2, b2, U1, u1, U2, u2,
           gamma, beta_ln, beta_vec):
    n, d = x.shape
    e = edge_index.shape[1]
    nl = W1.shape[0]
    bn = 2000
    rpt = n // 10

    src1 = edge_index[0]
    dst1 = edge_index[1]
    zeros_d = jnp.zeros((rpt, d), jnp.float32)
    zeros_c = jnp.zeros((rpt, CW), jnp.float32)
    ones_c = jnp.ones((K, CW), jnp.float32)

    cnt_k = _make_cnt(n, e)
    agg_k = _make_agg(n, e, d)
    pre_k = _make_pre(n, d, nl, bn)

    cntp = cnt_k(dst1, ones_c, zeros_c).reshape(NC, n, CW)
    z, msg, acc, inv, betas = pre_k(
        x, W_in, b_in.reshape(1, d),
        W1[0], b1[0].reshape(1, d), W2[0], b2[0].reshape(1, d),
        beta_vec.reshape(1, nl + 1), cntp)

    for l in range(nl):
        S = agg_k(msg, src1, dst1, zeros_d).reshape(NC, n, d)
        args = [z, S, msg, inv, acc, betas,
                U1[l], u1[l].reshape(1, d), U2[l], u2[l].reshape(1, d),
                gamma[l].reshape(1, d), beta_ln[l].reshape(1, d)]
        if l + 1 < nl:
            args += [W1[l + 1], b1[l + 1].reshape(1, d),
                     W2[l + 1], b2[l + 1].reshape(1, d)]
            z, acc, msg = _make_layer(n, d, nl, bn, l, True)(*args)
        else:
            z, acc = _make_layer(n, d, nl, bn, l, False)(*args)
    return acc


# speedup vs baseline: 4.8899x; 1.3393x over previous
import functools

import jax
import jax.numpy as jnp
from jax import lax
from jax.experimental import pallas as pl
from jax.experimental.pallas import tpu as pltpu
from jax.experimental.pallas import tpu_sc as plsc

NC = 2
NS = 16
K = 40
G = 5
KA = 40
CW = 128


def _make_agg(n, e, d):
    nw = NC * NS
    epw = e // nw
    groups = epw // (G * KA)
    nf = 10
    rpt = n // nf
    mesh = plsc.VectorSubcoreMesh(core_axis_name="c", subcore_axis_name="s")

    @functools.partial(
        pl.kernel,
        out_type=jax.ShapeDtypeStruct((NC * n, d), jnp.float32),
        mesh=mesh,
        scratch_types=[
            pltpu.VMEM_SHARED((n, d), jnp.float32),
            pltpu.VMEM((G * KA,), jnp.int32),
            [pltpu.VMEM((KA,), jnp.int32) for _ in range(G)],
            pltpu.VMEM((G, KA, d), jnp.float32),
            pltpu.SemaphoreType.DMA,
            pltpu.SemaphoreType.DMA,
        ],
    )
    def agg(msg_hbm, src_hbm, dst_hbm, zeros_hbm, out_hbm,
            acc_sh, sidx, didx, rows, semg, sems):
        cid = lax.axis_index("c")
        sid = lax.axis_index("s")
        wid = sid * NC + cid
        @pl.when(sid < nf)
        def _():
            pltpu.sync_copy(zeros_hbm, acc_sh.at[pl.ds(sid * rpt, rpt)])
        plsc.subcore_barrier()

        base = wid * epw

        def group_body(g, _):
            off = base + g * (G * KA)
            pltpu.sync_copy(src_hbm.at[pl.ds(off, G * KA)], sidx)

            @pl.when(g > 0)
            def _():
                for b in range(G):
                    pltpu.make_async_copy(rows.at[b], acc_sh.at[didx[b]],
                                          sems).wait()

            cps = [
                pltpu.async_copy(msg_hbm.at[sidx.at[pl.ds(b * KA, KA)]],
                                 rows.at[b], semg)
                for b in range(G)
            ]
            for b in range(G):
                pltpu.sync_copy(dst_hbm.at[pl.ds(off + b * KA, KA)], didx[b])
            for b in range(G):
                cps[b].wait()
                pltpu.async_copy(rows.at[b], acc_sh.at[didx[b]], sems,
                                 add=True)
            return 0

        lax.fori_loop(0, groups, group_body, 0)
        for b in range(G):
            pltpu.make_async_copy(rows.at[b], acc_sh.at[didx[b]], sems).wait()

        plsc.subcore_barrier()

        @pl.when(sid < nf)
        def _():
            pltpu.sync_copy(acc_sh.at[pl.ds(sid * rpt, rpt)],
                            out_hbm.at[pl.ds(cid * n + sid * rpt, rpt)])

    return agg


def _make_cnt(n, e):
    nw = NC * NS
    epw = e // nw
    groups = epw // (G * K)
    nf = 10
    rpt = n // nf
    mesh = plsc.VectorSubcoreMesh(core_axis_name="c", subcore_axis_name="s")

    @functools.partial(
        pl.kernel,
        out_type=jax.ShapeDtypeStruct((NC * n, CW), jnp.float32),
        mesh=mesh,
        scratch_types=[
            pltpu.VMEM_SHARED((n, CW), jnp.float32),
            [pltpu.VMEM((K,), jnp.int32) for _ in range(G)],
            pltpu.VMEM((K, CW), jnp.float32),
        ],
    )
    def cnt(dst_hbm, ones_hbm, zeros_hbm, out_hbm, acc_sh, didx, ones_v):
        cid = lax.axis_index("c")
        sid = lax.axis_index("s")
        wid = sid * NC + cid

        @pl.when(sid < nf)
        def _():
            pltpu.sync_copy(zeros_hbm, acc_sh.at[pl.ds(sid * rpt, rpt)])
        pltpu.sync_copy(ones_hbm, ones_v)
        plsc.subcore_barrier()

        base = wid * epw

        def group_body(g, _):
            off = base + g * (G * K)
            for b in range(G):
                pltpu.sync_copy(dst_hbm.at[pl.ds(off + b * K, K)], didx[b])
            for b in range(G):
                pltpu.sync_copy(ones_v, acc_sh.at[didx[b]], add=True)
            return 0

        lax.fori_loop(0, groups, group_body, 0)

        plsc.subcore_barrier()

        @pl.when(sid < nf)
        def _():
            pltpu.sync_copy(acc_sh.at[pl.ds(sid * rpt, rpt)],
                            out_hbm.at[pl.ds(cid * n + sid * rpt, rpt)])

    return cnt


def _dot(a, b):
    return jax.lax.dot_general(a, b, (((1,), (0,)), ((), ())),
                               preferred_element_type=jnp.float32)


def _pre_body(x_ref, win_ref, bin_ref, w1_ref, b1_ref, w2_ref, b2_ref,
              bv_ref, cntp_ref,
              z_ref, msg_ref, acc_ref, inv_ref, betas_ref):
    z = _dot(x_ref[...], win_ref[...]) + bin_ref[...]
    bv = bv_ref[...]
    ex = jnp.exp(bv - jnp.max(bv))
    betas = ex / jnp.sum(ex)
    betas_ref[...] = betas
    cnt = 1.0 + jnp.sum(cntp_ref[...][:, :, 0], axis=0)
    inv_ref[...] = (1.0 / cnt)[:, None]
    z_ref[...] = z
    acc_ref[...] = betas[0, 0] * z
    h = jnp.maximum(_dot(z, w1_ref[...]) + b1_ref[...], 0.0)
    msg_ref[...] = _dot(h, w2_ref[...]) + b2_ref[...]


def _layer_body(lidx, has_next,
                z_ref, s_ref, msg_ref, inv_ref, acc_ref, betas_ref,
                u1_ref, u1b_ref, u2_ref, u2b_ref, g_ref, bln_ref,
                *rest):
    if has_next:
        (w1_ref, b1_ref, w2_ref, b2_ref,
         zo_ref, acco_ref, msgo_ref) = rest
    else:
        zo_ref, acco_ref = rest
    sblk = s_ref[...]
    s = z_ref[...] + (sblk[0] + sblk[1] + msg_ref[...]) * inv_ref[...]
    h2 = jnp.maximum(_dot(s, u1_ref[...]) + u1b_ref[...], 0.0)
    o = _dot(h2, u2_ref[...]) + u2b_ref[...]
    mu = jnp.mean(o, axis=-1, keepdims=True)
    var = jnp.mean((o - mu) ** 2, axis=-1, keepdims=True)
    zn = (o - mu) * lax.rsqrt(var + 1e-5) * g_ref[...] + bln_ref[...]
    zo_ref[...] = zn
    acco_ref[...] = acc_ref[...] + betas_ref[0, lidx + 1] * zn
    if has_next:
        h = jnp.maximum(_dot(zn, w1_ref[...]) + b1_ref[...], 0.0)
        msgo_ref[...] = _dot(h, w2_ref[...]) + b2_ref[...]


def _full(shape):
    nd = len(shape)
    return pl.BlockSpec(shape, lambda i: (0,) * nd)


def _make_pre(n, d, nl, bn):
    grid = (n // bn,)
    row = pl.BlockSpec((bn, d), lambda i: (i, 0))
    return pl.pallas_call(
        _pre_body,
        grid=grid,
        in_specs=[
            row, _full((d, d)), _full((1, d)),
            _full((d, d)), _full((1, d)), _full((d, d)), _full((1, d)),
            _full((1, nl + 1)),
            pl.BlockSpec((NC, bn, CW), lambda i: (0, i, 0)),
        ],
        out_specs=[row, row, row,
                   pl.BlockSpec((bn, 1), lambda i: (i, 0)),
                   _full((1, nl + 1))],
        out_shape=[
            jax.ShapeDtypeStruct((n, d), jnp.float32),
            jax.ShapeDtypeStruct((n, d), jnp.float32),
            jax.ShapeDtypeStruct((n, d), jnp.float32),
            jax.ShapeDtypeStruct((n, 1), jnp.float32),
            jax.ShapeDtypeStruct((1, nl + 1), jnp.float32),
        ],
    )


def _make_layer(n, d, nl, bn, lidx, has_next):
    grid = (n // bn,)
    row = pl.BlockSpec((bn, d), lambda i: (i, 0))
    in_specs = [
        row,
        pl.BlockSpec((NC, bn, d), lambda i: (0, i, 0)),
        row,
        pl.BlockSpec((bn, 1), lambda i: (i, 0)),
        row,
        _full((1, nl + 1)),
        _full((d, d)), _full((1, d)), _full((d, d)), _full((1, d)),
        _full((1, d)), _full((1, d)),
    ]
    out_specs = [row, row]
    out_shape = [
        jax.ShapeDtypeStruct((n, d), jnp.float32),
        jax.ShapeDtypeStruct((n, d), jnp.float32),
    ]
    if has_next:
        in_specs += [_full((d, d)), _full((1, d)), _full((d, d)), _full((1, d))]
        out_specs.append(row)
        out_shape.append(jax.ShapeDtypeStruct((n, d), jnp.float32))
    return pl.pallas_call(
        functools.partial(_layer_body, lidx, has_next),
        grid=grid,
        in_specs=in_specs,
        out_specs=out_specs,
        out_shape=out_shape,
    )


def kernel(x, edge_index, W_in, b_in, W1, b1, W2, b2, U1, u1, U2, u2,
           gamma, beta_ln, beta_vec):
    n, d = x.shape
    e = edge_index.shape[1]
    nl = W1.shape[0]
    bn = 2000
    rpt = n // 10

    src1 = edge_index[0]
    dst1 = edge_index[1]
    zeros_d = jnp.zeros((rpt, d), jnp.float32)
    zeros_c = jnp.zeros((rpt, CW), jnp.float32)
    ones_c = jnp.ones((K, CW), jnp.float32)

    cnt_k = _make_cnt(n, e)
    agg_k = _make_agg(n, e, d)
    pre_k = _make_pre(n, d, nl, bn)

    cntp = cnt_k(dst1, ones_c, zeros_c).reshape(NC, n, CW)
    z, msg, acc, inv, betas = pre_k(
        x, W_in, b_in.reshape(1, d),
        W1[0], b1[0].reshape(1, d), W2[0], b2[0].reshape(1, d),
        beta_vec.reshape(1, nl + 1), cntp)

    for l in range(nl):
        S = agg_k(msg, src1, dst1, zeros_d).reshape(NC, n, d)
        args = [z, S, msg, inv, acc, betas,
                U1[l], u1[l].reshape(1, d), U2[l], u2[l].reshape(1, d),
                gamma[l].reshape(1, d), beta_ln[l].reshape(1, d)]
        if l + 1 < nl:
            args += [W1[l + 1], b1[l + 1].reshape(1, d),
                     W2[l + 1], b2[l + 1].reshape(1, d)]
            z, acc, msg = _make_layer(n, d, nl, bn, l, True)(*args)
        else:
            z, acc = _make_layer(n, d, nl, bn, l, False)(*args)
    return acc


# speedup vs baseline: 6.4666x; 1.3224x over previous
import functools

import jax
import jax.numpy as jnp
from jax import lax
from jax.experimental import pallas as pl
from jax.experimental.pallas import tpu as pltpu
from jax.experimental.pallas import tpu_sc as plsc

NC = 2
NS = 16
K = 40
G = 5
KA = 40
CW = 128


def _make_agg(n, e, d):
    nw = NC * NS
    epw = e // nw
    groups = epw // (G * KA)
    nf = 10
    rpt = n // nf
    mesh = plsc.VectorSubcoreMesh(core_axis_name="c", subcore_axis_name="s")

    @functools.partial(
        pl.kernel,
        out_type=jax.ShapeDtypeStruct((NC * n, d), jnp.float32),
        mesh=mesh,
        scratch_types=[
            pltpu.VMEM_SHARED((n, d), jnp.float32),
            [pltpu.VMEM((G * KA,), jnp.int32) for _ in range(2)],
            [[pltpu.VMEM((KA,), jnp.int32) for _ in range(G)]
             for _ in range(2)],
            pltpu.VMEM((G, KA, d), jnp.float32),
            pltpu.SemaphoreType.DMA,
            pltpu.SemaphoreType.DMA,
            pltpu.SemaphoreType.DMA,
        ],
    )
    def agg(msg_hbm, src_hbm, dst_hbm, zeros_hbm, out_hbm,
            acc_sh, sidx2, didx2, rows, semg, sems, semi):
        cid = lax.axis_index("c")
        sid = lax.axis_index("s")
        wid = sid * NC + cid
        @pl.when(sid < nf)
        def _():
            pltpu.sync_copy(zeros_hbm, acc_sh.at[pl.ds(sid * rpt, rpt)])
        plsc.subcore_barrier()

        base = wid * epw

        def issue_idx(g, p):
            off = base + g * (G * KA)
            pltpu.async_copy(src_hbm.at[pl.ds(off, G * KA)], sidx2[p], semi)
            for b in range(G):
                pltpu.async_copy(dst_hbm.at[pl.ds(off + b * KA, KA)],
                                 didx2[p][b], semi)

        def wait_idx(g, p):
            off = base + g * (G * KA)
            pltpu.make_async_copy(src_hbm.at[pl.ds(off, G * KA)], sidx2[p],
                                  semi).wait()
            for b in range(G):
                pltpu.make_async_copy(dst_hbm.at[pl.ds(off + b * KA, KA)],
                                      didx2[p][b], semi).wait()

        issue_idx(0, 0)

        def group_body(g, p, first):
            sidx, didx = sidx2[p], didx2[p]
            if not first:
                for b in range(G):
                    pltpu.make_async_copy(rows.at[b],
                                          acc_sh.at[didx2[1 - p][b]],
                                          sems).wait()
            wait_idx(g, p)
            cps = [
                pltpu.async_copy(msg_hbm.at[sidx.at[pl.ds(b * KA, KA)]],
                                 rows.at[b], semg)
                for b in range(G)
            ]
            if isinstance(g, int):
                if g + 1 < groups:
                    issue_idx(g + 1, 1 - p)
            else:
                @pl.when(g + 1 < groups)
                def _():
                    issue_idx(g + 1, 1 - p)
            for b in range(G):
                cps[b].wait()
                pltpu.async_copy(rows.at[b], acc_sh.at[didx[b]], sems,
                                 add=True)

        group_body(0, 0, True)
        group_body(1, 1, False)

        def outer(i, _):
            group_body(i * 2, 0, False)
            group_body(i * 2 + 1, 1, False)
            return 0

        lax.fori_loop(1, groups // 2, outer, 0)

        for b in range(G):
            pltpu.make_async_copy(rows.at[b], acc_sh.at[didx2[1][b]],
                                  sems).wait()

        plsc.subcore_barrier()

        @pl.when(sid < nf)
        def _():
            pltpu.sync_copy(acc_sh.at[pl.ds(sid * rpt, rpt)],
                            out_hbm.at[pl.ds(cid * n + sid * rpt, rpt)])

    return agg


def _make_cnt(n, e):
    nw = NC * NS
    epw = e // nw
    groups = epw // (G * K)
    nf = 10
    rpt = n // nf
    mesh = plsc.VectorSubcoreMesh(core_axis_name="c", subcore_axis_name="s")

    @functools.partial(
        pl.kernel,
        out_type=jax.ShapeDtypeStruct((NC * n, CW), jnp.float32),
        mesh=mesh,
        scratch_types=[
            pltpu.VMEM_SHARED((n, CW), jnp.float32),
            [pltpu.VMEM((K,), jnp.int32) for _ in range(G)],
            pltpu.VMEM((K, CW), jnp.float32),
        ],
    )
    def cnt(dst_hbm, ones_hbm, zeros_hbm, out_hbm, acc_sh, didx, ones_v):
        cid = lax.axis_index("c")
        sid = lax.axis_index("s")
        wid = sid * NC + cid

        @pl.when(sid < nf)
        def _():
            pltpu.sync_copy(zeros_hbm, acc_sh.at[pl.ds(sid * rpt, rpt)])
        pltpu.sync_copy(ones_hbm, ones_v)
        plsc.subcore_barrier()

        base = wid * epw

        def group_body(g, _):
            off = base + g * (G * K)
            for b in range(G):
                pltpu.sync_copy(dst_hbm.at[pl.ds(off + b * K, K)], didx[b])
            for b in range(G):
                pltpu.sync_copy(ones_v, acc_sh.at[didx[b]], add=True)
            return 0

        lax.fori_loop(0, groups, group_body, 0)

        plsc.subcore_barrier()

        @pl.when(sid < nf)
        def _():
            pltpu.sync_copy(acc_sh.at[pl.ds(sid * rpt, rpt)],
                            out_hbm.at[pl.ds(cid * n + sid * rpt, rpt)])

    return cnt


def _dot(a, b):
    return jax.lax.dot_general(a, b, (((1,), (0,)), ((), ())),
                               preferred_element_type=jnp.float32)


def _pre_body(x_ref, win_ref, bin_ref, w1_ref, b1_ref, w2_ref, b2_ref,
              bv_ref, cntp_ref,
              z_ref, msg_ref, acc_ref, inv_ref, betas_ref):
    z = _dot(x_ref[...], win_ref[...]) + bin_ref[...]
    bv = bv_ref[...]
    ex = jnp.exp(bv - jnp.max(bv))
    betas = ex / jnp.sum(ex)
    betas_ref[...] = betas
    cnt = 1.0 + jnp.sum(cntp_ref[...][:, :, 0], axis=0)
    inv_ref[...] = (1.0 / cnt)[:, None]
    z_ref[...] = z
    acc_ref[...] = betas[0, 0] * z
    h = jnp.maximum(_dot(z, w1_ref[...]) + b1_ref[...], 0.0)
    msg_ref[...] = _dot(h, w2_ref[...]) + b2_ref[...]


def _layer_body(lidx, has_next,
                z_ref, s_ref, msg_ref, inv_ref, acc_ref, betas_ref,
                u1_ref, u1b_ref, u2_ref, u2b_ref, g_ref, bln_ref,
                *rest):
    if has_next:
        (w1_ref, b1_ref, w2_ref, b2_ref,
         zo_ref, acco_ref, msgo_ref) = rest
    else:
        zo_ref, acco_ref = rest
    sblk = s_ref[...]
    s = z_ref[...] + (sblk[0] + sblk[1] + msg_ref[...]) * inv_ref[...]
    h2 = jnp.maximum(_dot(s, u1_ref[...]) + u1b_ref[...], 0.0)
    o = _dot(h2, u2_ref[...]) + u2b_ref[...]
    mu = jnp.mean(o, axis=-1, keepdims=True)
    var = jnp.mean((o - mu) ** 2, axis=-1, keepdims=True)
    zn = (o - mu) * lax.rsqrt(var + 1e-5) * g_ref[...] + bln_ref[...]
    zo_ref[...] = zn
    acco_ref[...] = acc_ref[...] + betas_ref[0, lidx + 1] * zn
    if has_next:
        h = jnp.maximum(_dot(zn, w1_ref[...]) + b1_ref[...], 0.0)
        msgo_ref[...] = _dot(h, w2_ref[...]) + b2_ref[...]


def _full(shape):
    nd = len(shape)
    return pl.BlockSpec(shape, lambda i: (0,) * nd)


def _make_pre(n, d, nl, bn):
    grid = (n // bn,)
    row = pl.BlockSpec((bn, d), lambda i: (i, 0))
    return pl.pallas_call(
        _pre_body,
        grid=grid,
        in_specs=[
            row, _full((d, d)), _full((1, d)),
            _full((d, d)), _full((1, d)), _full((d, d)), _full((1, d)),
            _full((1, nl + 1)),
            pl.BlockSpec((NC, bn, CW), lambda i: (0, i, 0)),
        ],
        out_specs=[row, row, row,
                   pl.BlockSpec((bn, 1), lambda i: (i, 0)),
                   _full((1, nl + 1))],
        out_shape=[
            jax.ShapeDtypeStruct((n, d), jnp.float32),
            jax.ShapeDtypeStruct((n, d), jnp.float32),
            jax.ShapeDtypeStruct((n, d), jnp.float32),
            jax.ShapeDtypeStruct((n, 1), jnp.float32),
            jax.ShapeDtypeStruct((1, nl + 1), jnp.float32),
        ],
    )


def _make_layer(n, d, nl, bn, lidx, has_next):
    grid = (n // bn,)
    row = pl.BlockSpec((bn, d), lambda i: (i, 0))
    in_specs = [
        row,
        pl.BlockSpec((NC, bn, d), lambda i: (0, i, 0)),
        row,
        pl.BlockSpec((bn, 1), lambda i: (i, 0)),
        row,
        _full((1, nl + 1)),
        _full((d, d)), _full((1, d)), _full((d, d)), _full((1, d)),
        _full((1, d)), _full((1, d)),
    ]
    out_specs = [row, row]
    out_shape = [
        jax.ShapeDtypeStruct((n, d), jnp.float32),
        jax.ShapeDtypeStruct((n, d), jnp.float32),
    ]
    if has_next:
        in_specs += [_full((d, d)), _full((1, d)), _full((d, d)), _full((1, d))]
        out_specs.append(row)
        out_shape.append(jax.ShapeDtypeStruct((n, d), jnp.float32))
    return pl.pallas_call(
        functools.partial(_layer_body, lidx, has_next),
        grid=grid,
        in_specs=in_specs,
        out_specs=out_specs,
        out_shape=out_shape,
    )


def kernel(x, edge_index, W_in, b_in, W1, b1, W2, b2, U1, u1, U2, u2,
           gamma, beta_ln, beta_vec):
    n, d = x.shape
    e = edge_index.shape[1]
    nl = W1.shape[0]
    bn = 2000
    rpt = n // 10

    src1 = edge_index[0]
    dst1 = edge_index[1]
    zeros_d = jnp.zeros((rpt, d), jnp.float32)
    zeros_c = jnp.zeros((rpt, CW), jnp.float32)
    ones_c = jnp.ones((K, CW), jnp.float32)

    cnt_k = _make_cnt(n, e)
    agg_k = _make_agg(n, e, d)
    pre_k = _make_pre(n, d, nl, bn)

    cntp = cnt_k(dst1, ones_c, zeros_c).reshape(NC, n, CW)
    z, msg, acc, inv, betas = pre_k(
        x, W_in, b_in.reshape(1, d),
        W1[0], b1[0].reshape(1, d), W2[0], b2[0].reshape(1, d),
        beta_vec.reshape(1, nl + 1), cntp)

    for l in range(nl):
        S = agg_k(msg, src1, dst1, zeros_d).reshape(NC, n, d)
        args = [z, S, msg, inv, acc, betas,
                U1[l], u1[l].reshape(1, d), U2[l], u2[l].reshape(1, d),
                gamma[l].reshape(1, d), beta_ln[l].reshape(1, d)]
        if l + 1 < nl:
            args += [W1[l + 1], b1[l + 1].reshape(1, d),
                     W2[l + 1], b2[l + 1].reshape(1, d)]
            z, acc, msg = _make_layer(n, d, nl, bn, l, True)(*args)
        else:
            z, acc = _make_layer(n, d, nl, bn, l, False)(*args)
    return acc
